# initial kernel scaffold (unmeasured)
import jax
import jax.numpy as jnp
from jax import lax
from jax.experimental import pallas as pl
from jax.experimental.pallas import tpu as pltpu

N_DEV = 4
M = 512
K = 1024
NL = 8192
NG = N_DEV * NL


def _local_logits(x, W):
    TJ = 2048

    def body(x_ref, w_ref, o_ref):
        xb = x_ref[...].astype(jnp.bfloat16)
        wb = w_ref[...].astype(jnp.bfloat16)
        o_ref[...] = jnp.dot(
            xb, wb, preferred_element_type=jnp.float32
        ).astype(jnp.bfloat16)

    return pl.pallas_call(
        body,
        grid=(NL // TJ,),
        in_specs=[
            pl.BlockSpec((M, K), lambda j: (0, 0)),
            pl.BlockSpec((K, TJ), lambda j: (0, j)),
        ],
        out_specs=pl.BlockSpec((M, TJ), lambda j: (0, j)),
        out_shape=jax.ShapeDtypeStruct((M, NL), jnp.bfloat16),
    )(x, W)


def _allgather_softmax(lg):

    def body(lg_ref, out_ref, send_sems, recv_sems):
        my = lax.axis_index("i")
        left = lax.rem(my + (N_DEV - 1), N_DEV)
        right = lax.rem(my + 1, N_DEV)

        barrier_sem = pltpu.get_barrier_semaphore()
        for nbr in (left, right):
            pl.semaphore_signal(
                barrier_sem, inc=1,
                device_id=(nbr,), device_id_type=pl.DeviceIdType.MESH,
            )
        pl.semaphore_wait(barrier_sem, 2)

        out_ref[:, pl.ds(my * NL, NL)] = lg_ref[...]

        for h in range(N_DEV - 1):
            origin = lax.rem(my + (N_DEV - h), N_DEV)
            col = pl.multiple_of(origin * NL, NL)
            rdma = pltpu.make_async_remote_copy(
                src_ref=out_ref.at[:, pl.ds(col, NL)],
                dst_ref=out_ref.at[:, pl.ds(col, NL)],
                send_sem=send_sems.at[h],
                recv_sem=recv_sems.at[h],
                device_id=(right,),
                device_id_type=pl.DeviceIdType.MESH,
            )
            rdma.start()
            rdma.wait()

        TN = 2048
        m = jnp.full((M, 1), -jnp.inf, jnp.float32)
        for t in range(NG // TN):
            l = out_ref[:, t * TN:(t + 1) * TN].astype(jnp.float32)
            m = jnp.maximum(m, jnp.max(l, axis=1, keepdims=True))
        s = jnp.zeros((M, 1), jnp.float32)
        for t in range(NG // TN):
            l = out_ref[:, t * TN:(t + 1) * TN].astype(jnp.float32)
            e = jnp.exp(l - m)
            s = s + jnp.sum(e, axis=1, keepdims=True)
            out_ref[:, t * TN:(t + 1) * TN] = e.astype(jnp.bfloat16)
        r = 1.0 / s
        for t in range(NG // TN):
            e = out_ref[:, t * TN:(t + 1) * TN].astype(jnp.float32)
            out_ref[:, t * TN:(t + 1) * TN] = (e * r).astype(jnp.bfloat16)

    return pl.pallas_call(
        body,
        out_shape=jax.ShapeDtypeStruct((M, NG), jnp.bfloat16),
        in_specs=[pl.BlockSpec(memory_space=pltpu.VMEM)],
        out_specs=pl.BlockSpec(memory_space=pltpu.VMEM),
        scratch_shapes=[
            pltpu.SemaphoreType.DMA((N_DEV - 1,)),
            pltpu.SemaphoreType.DMA((N_DEV - 1,)),
        ],
        compiler_params=pltpu.CompilerParams(collective_id=0),
    )(lg)


def kernel(x, W):
    lg = _local_logits(x, W)
    return _allgather_softmax(lg)


# baseline (device time: 357949 ns/iter reference)
import jax
import jax.numpy as jnp
from jax import lax
from jax.experimental import pallas as pl
from jax.experimental.pallas import tpu as pltpu

N_DEV = 4
M = 512
K = 1024
NL = 8192
NG = N_DEV * NL


def _local_logits(x, W):
    TJ = 2048

    def body(x_ref, w_ref, o_ref):
        xb = x_ref[...].astype(jnp.bfloat16)
        wb = w_ref[...].astype(jnp.bfloat16)
        o_ref[...] = jnp.dot(
            xb, wb, preferred_element_type=jnp.float32
        ).astype(jnp.bfloat16)

    return pl.pallas_call(
        body,
        grid=(NL // TJ,),
        in_specs=[
            pl.BlockSpec((M, K), lambda j: (0, 0)),
            pl.BlockSpec((K, TJ), lambda j: (0, j)),
        ],
        out_specs=pl.BlockSpec((M, TJ), lambda j: (0, j)),
        out_shape=jax.ShapeDtypeStruct((M, NL), jnp.bfloat16),
    )(x, W)


def _allgather_softmax(lg):

    def body(lg_ref, out_ref, send_sems, recv_sems):
        my = lax.axis_index("i")
        left = lax.rem(my + (N_DEV - 1), N_DEV)
        right = lax.rem(my + 1, N_DEV)

        barrier_sem = pltpu.get_barrier_semaphore()
        for nbr in (left, right):
            pl.semaphore_signal(
                barrier_sem, inc=1,
                device_id=(nbr,), device_id_type=pl.DeviceIdType.MESH,
            )
        pl.semaphore_wait(barrier_sem, 2)

        out_ref[:, pl.ds(my * NL, NL)] = lg_ref[...]

        for h in range(N_DEV - 1):
            origin = lax.rem(my + (N_DEV - h), N_DEV)
            col = pl.multiple_of(origin * NL, NL)
            rdma = pltpu.make_async_remote_copy(
                src_ref=out_ref.at[:, pl.ds(col, NL)],
                dst_ref=out_ref.at[:, pl.ds(col, NL)],
                send_sem=send_sems.at[h],
                recv_sem=recv_sems.at[h],
                device_id=(right,),
                device_id_type=pl.DeviceIdType.MESH,
            )
            rdma.start()
            rdma.wait()

        TN = 2048
        n_tiles = NG // TN

        def max_body(t, m):
            l = out_ref[:, pl.ds(t * TN, TN)].astype(jnp.float32)
            return jnp.maximum(m, jnp.max(l, axis=1, keepdims=True))

        m = lax.fori_loop(
            0, n_tiles, max_body, jnp.full((M, 1), -jnp.inf, jnp.float32)
        )

        def exp_body(t, s):
            l = out_ref[:, pl.ds(t * TN, TN)].astype(jnp.float32)
            e = jnp.exp(l - m)
            out_ref[:, pl.ds(t * TN, TN)] = e.astype(jnp.bfloat16)
            return s + jnp.sum(e, axis=1, keepdims=True)

        s = lax.fori_loop(0, n_tiles, exp_body, jnp.zeros((M, 1), jnp.float32))
        r = 1.0 / s

        def norm_body(t, carry):
            e = out_ref[:, pl.ds(t * TN, TN)].astype(jnp.float32)
            out_ref[:, pl.ds(t * TN, TN)] = (e * r).astype(jnp.bfloat16)
            return carry

        lax.fori_loop(0, n_tiles, norm_body, jnp.zeros((1, 1), jnp.float32))

    return pl.pallas_call(
        body,
        out_shape=jax.ShapeDtypeStruct((M, NG), jnp.bfloat16),
        in_specs=[pl.BlockSpec(memory_space=pltpu.VMEM)],
        out_specs=pl.BlockSpec(memory_space=pltpu.VMEM),
        scratch_shapes=[
            pltpu.SemaphoreType.DMA((N_DEV - 1,)),
            pltpu.SemaphoreType.DMA((N_DEV - 1,)),
        ],
        compiler_params=pltpu.CompilerParams(
            collective_id=0, vmem_limit_bytes=60 * 1024 * 1024
        ),
    )(lg)


def kernel(x, W):
    lg = _local_logits(x, W)
    return _allgather_softmax(lg)


# device time: 213491 ns/iter; 1.6766x vs baseline; 1.6766x over previous
import jax
import jax.numpy as jnp
from jax import lax
from jax.experimental import pallas as pl
from jax.experimental.pallas import tpu as pltpu

N_DEV = 4
M = 512
K = 1024
NL = 8192
NG = N_DEV * NL


def _local_logits(x, W):
    TJ = 2048

    def body(x_ref, w_ref, o_ref):
        xb = x_ref[...].astype(jnp.bfloat16)
        wb = w_ref[...].astype(jnp.bfloat16)
        o_ref[...] = jnp.dot(
            xb, wb, preferred_element_type=jnp.float32
        ).astype(jnp.bfloat16)

    return pl.pallas_call(
        body,
        grid=(NL // TJ,),
        in_specs=[
            pl.BlockSpec((M, K), lambda j: (0, 0)),
            pl.BlockSpec((K, TJ), lambda j: (0, j)),
        ],
        out_specs=pl.BlockSpec((M, TJ), lambda j: (0, j)),
        out_shape=jax.ShapeDtypeStruct((M, NL), jnp.bfloat16),
    )(x, W)


def _allgather_softmax(lg):

    TN = 2048
    MH = M // 2

    def body(lg_ref, out_ref, send_sems, recv_sems):
        my = lax.axis_index("i")
        left = lax.rem(my + (N_DEV - 1), N_DEV)
        right = lax.rem(my + 1, N_DEV)

        barrier_sem = pltpu.get_barrier_semaphore()
        for nbr in (left, right):
            pl.semaphore_signal(
                barrier_sem, inc=1,
                device_id=(nbr,), device_id_type=pl.DeviceIdType.MESH,
            )
        pl.semaphore_wait(barrier_sem, 2)

        def remote_copy(src, dst, sem_idx, dst_dev):
            return pltpu.make_async_remote_copy(
                src_ref=src,
                dst_ref=dst,
                send_sem=send_sems.at[sem_idx],
                recv_sem=recv_sems.at[sem_idx],
                device_id=(dst_dev,),
                device_id_type=pl.DeviceIdType.MESH,
            )

        def cols(origin):
            return pl.ds(pl.multiple_of(origin * NL, NL), NL)

        r1 = remote_copy(lg_ref.at[:, :], out_ref.at[:, cols(my)], 0, right)
        l1 = remote_copy(lg_ref.at[:, :], out_ref.at[:, cols(my)], 1, left)
        r1.start()
        l1.start()

        out_ref[:, cols(my)] = lg_ref[...]

        def chunk_stats(col0, ms):
            def tbody(t, carry):
                m, s = carry
                l = out_ref[:, pl.ds(col0 + t * TN, TN)].astype(jnp.float32)
                tm = jnp.max(l, axis=1, keepdims=True)
                te = jnp.exp(l - tm)
                ts = jnp.sum(te, axis=1, keepdims=True)
                nm = jnp.maximum(m, tm)
                return nm, s * jnp.exp(m - nm) + ts * jnp.exp(tm - nm)
            return lax.fori_loop(0, NL // TN, tbody, ms)

        ms = (
            jnp.full((M, 1), -jnp.inf, jnp.float32),
            jnp.zeros((M, 1), jnp.float32),
        )
        ms = chunk_stats(my * NL, ms)

        r1.wait_recv()
        r2 = remote_copy(
            out_ref.at[pl.ds(0, MH), cols(left)],
            out_ref.at[pl.ds(0, MH), cols(left)],
            2, right,
        )
        r2.start()
        l1.wait_recv()
        l2 = remote_copy(
            out_ref.at[pl.ds(MH, MH), cols(right)],
            out_ref.at[pl.ds(MH, MH), cols(right)],
            3, left,
        )
        l2.start()
        r1.wait_send()
        l1.wait_send()

        ms = chunk_stats(left * NL, ms)
        ms = chunk_stats(right * NL, ms)

        r2.wait_recv()
        l2.wait_recv()
        r2.wait_send()
        l2.wait_send()
        opp = lax.rem(my + 2, N_DEV)
        ms = chunk_stats(opp * NL, ms)

        m, s = ms
        r = 1.0 / s

        def norm_body(t, carry):
            l = out_ref[:, pl.ds(t * TN, TN)].astype(jnp.float32)
            out_ref[:, pl.ds(t * TN, TN)] = (
                jnp.exp(l - m) * r
            ).astype(jnp.bfloat16)
            return carry

        lax.fori_loop(0, NG // TN, norm_body, jnp.zeros((1, 1), jnp.float32))

    return pl.pallas_call(
        body,
        out_shape=jax.ShapeDtypeStruct((M, NG), jnp.bfloat16),
        in_specs=[pl.BlockSpec(memory_space=pltpu.VMEM)],
        out_specs=pl.BlockSpec(memory_space=pltpu.VMEM),
        scratch_shapes=[
            pltpu.SemaphoreType.DMA((4,)),
            pltpu.SemaphoreType.DMA((4,)),
        ],
        compiler_params=pltpu.CompilerParams(
            collective_id=0, vmem_limit_bytes=60 * 1024 * 1024
        ),
    )(lg)


def kernel(x, W):
    lg = _local_logits(x, W)
    return _allgather_softmax(lg)


# device time: 204155 ns/iter; 1.7533x vs baseline; 1.0457x over previous
import jax
import jax.numpy as jnp
from jax import lax
from jax.experimental import pallas as pl
from jax.experimental.pallas import tpu as pltpu

N_DEV = 4
M = 512
K = 1024
NL = 8192
NG = N_DEV * NL


def _local_logits(x, W):
    TJ = 2048

    def body(x_ref, w_ref, o_ref):
        xb = x_ref[...].astype(jnp.bfloat16)
        wb = w_ref[...].astype(jnp.bfloat16)
        o_ref[...] = jnp.dot(
            xb, wb, preferred_element_type=jnp.float32
        ).astype(jnp.bfloat16)

    return pl.pallas_call(
        body,
        grid=(NL // TJ,),
        in_specs=[
            pl.BlockSpec((M, K), lambda j: (0, 0)),
            pl.BlockSpec((K, TJ), lambda j: (0, j)),
        ],
        out_specs=pl.BlockSpec((M, TJ), lambda j: (0, j)),
        out_shape=jax.ShapeDtypeStruct((M, NL), jnp.bfloat16),
    )(x, W)


def _allgather_softmax(lg):

    TN = 2048
    MH = M // 2

    def body(lg_ref, out_ref, comm_ref, send_sems, recv_sems, copy_sem):
        my = lax.axis_index("i")
        left = lax.rem(my + (N_DEV - 1), N_DEV)
        right = lax.rem(my + 1, N_DEV)

        barrier_sem = pltpu.get_barrier_semaphore()
        for nbr in (left, right):
            pl.semaphore_signal(
                barrier_sem, inc=1,
                device_id=(nbr,), device_id_type=pl.DeviceIdType.MESH,
            )
        pl.semaphore_wait(barrier_sem, 2)

        def remote_copy(src, dst, sem_idx, dst_dev):
            return pltpu.make_async_remote_copy(
                src_ref=src,
                dst_ref=dst,
                send_sem=send_sems.at[sem_idx],
                recv_sem=recv_sems.at[sem_idx],
                device_id=(dst_dev,),
                device_id_type=pl.DeviceIdType.MESH,
            )

        def cols(origin):
            return pl.ds(pl.multiple_of(origin * NL, NL), NL)

        r1 = remote_copy(lg_ref.at[:, :], comm_ref.at[:, cols(my)], 0, right)
        l1 = remote_copy(lg_ref.at[:, :], comm_ref.at[:, cols(my)], 1, left)
        r1.start()
        l1.start()

        comm_ref[:, cols(my)] = lg_ref[...]

        def chunk_stats(col0, ms):
            def tbody(t, carry):
                m, s = carry
                l = comm_ref[:, pl.ds(col0 + t * TN, TN)].astype(jnp.float32)
                tm = jnp.max(l, axis=1, keepdims=True)
                te = jnp.exp(l - tm)
                ts = jnp.sum(te, axis=1, keepdims=True)
                nm = jnp.maximum(m, tm)
                return nm, s * jnp.exp(m - nm) + ts * jnp.exp(tm - nm)
            return lax.fori_loop(0, NL // TN, tbody, ms)

        ms = (
            jnp.full((M, 1), -jnp.inf, jnp.float32),
            jnp.zeros((M, 1), jnp.float32),
        )
        ms = chunk_stats(my * NL, ms)

        r1.wait_recv()
        r2 = remote_copy(
            comm_ref.at[pl.ds(0, MH), cols(left)],
            comm_ref.at[pl.ds(0, MH), cols(left)],
            2, right,
        )
        r2.start()
        l1.wait_recv()
        l2 = remote_copy(
            comm_ref.at[pl.ds(MH, MH), cols(right)],
            comm_ref.at[pl.ds(MH, MH), cols(right)],
            3, left,
        )
        l2.start()
        r1.wait_send()
        l1.wait_send()

        ms = chunk_stats(left * NL, ms)
        ms = chunk_stats(right * NL, ms)

        r2.wait_recv()
        l2.wait_recv()
        r2.wait_send()
        l2.wait_send()
        opp = lax.rem(my + 2, N_DEV)
        ms = chunk_stats(opp * NL, ms)

        m, s = ms
        r = 1.0 / s

        def out_dma(t):
            return pltpu.make_async_copy(
                comm_ref.at[:, pl.ds(t * TN, TN)],
                out_ref.at[:, pl.ds(t * TN, TN)],
                copy_sem,
            )

        def norm_body(t, carry):
            l = comm_ref[:, pl.ds(t * TN, TN)].astype(jnp.float32)
            comm_ref[:, pl.ds(t * TN, TN)] = (
                jnp.exp(l - m) * r
            ).astype(jnp.bfloat16)
            out_dma(t).start()
            return carry

        lax.fori_loop(0, NG // TN, norm_body, jnp.zeros((1, 1), jnp.float32))

        def drain_body(t, carry):
            out_dma(t).wait()
            return carry

        lax.fori_loop(0, NG // TN, drain_body, jnp.zeros((1, 1), jnp.float32))

    return pl.pallas_call(
        body,
        out_shape=jax.ShapeDtypeStruct((M, NG), jnp.bfloat16),
        in_specs=[pl.BlockSpec(memory_space=pltpu.VMEM)],
        out_specs=pl.BlockSpec(memory_space=pl.ANY),
        scratch_shapes=[
            pltpu.VMEM((M, NG), jnp.bfloat16),
            pltpu.SemaphoreType.DMA((4,)),
            pltpu.SemaphoreType.DMA((4,)),
            pltpu.SemaphoreType.DMA,
        ],
        compiler_params=pltpu.CompilerParams(
            collective_id=0, vmem_limit_bytes=60 * 1024 * 1024
        ),
    )(lg)


def kernel(x, W):
    lg = _local_logits(x, W)
    return _allgather_softmax(lg)


# device time: 185842 ns/iter; 1.9261x vs baseline; 1.0985x over previous
import jax
import jax.numpy as jnp
from jax import lax
from jax.experimental import pallas as pl
from jax.experimental.pallas import tpu as pltpu

N_DEV = 4
M = 512
K = 1024
NL = 8192
NG = N_DEV * NL
TW = 1024
NT = NL // TW
TN = 2048
MH = M // 2


def _fused(x, W):
    def body(x_ref, w_hbm, out_ref, comm_ref, wt_ref,
             w_sems, send_sems, recv_sems, s2_send, s2_recv, copy_sem):
        my = lax.axis_index("i")
        left = lax.rem(my + (N_DEV - 1), N_DEV)
        right = lax.rem(my + 1, N_DEV)

        barrier_sem = pltpu.get_barrier_semaphore()
        for nbr in (left, right):
            pl.semaphore_signal(
                barrier_sem, inc=1,
                device_id=(nbr,), device_id_type=pl.DeviceIdType.MESH,
            )
        pl.semaphore_wait(barrier_sem, 2)

        def remote_copy(src, dst, sends, recvs, idx, dst_dev):
            return pltpu.make_async_remote_copy(
                src_ref=src,
                dst_ref=dst,
                send_sem=sends.at[idx],
                recv_sem=recvs.at[idx],
                device_id=(dst_dev,),
                device_id_type=pl.DeviceIdType.MESH,
            )

        def cols(origin):
            return pl.ds(pl.multiple_of(origin * NL, NL), NL)

        def sub_cols(origin, j):
            return pl.ds(pl.multiple_of(origin * NL + j * TW, TW), TW)

        def w_dma(j):
            return pltpu.make_async_copy(
                w_hbm.at[:, pl.ds(j * TW, TW)],
                wt_ref.at[j % 2],
                w_sems.at[j % 2],
            )

        w_dma(0).start()
        xb = x_ref[...].astype(jnp.bfloat16)
        hop1 = []
        for j in range(NT):
            if j + 1 < NT:
                w_dma(j + 1).start()
            w_dma(j).wait()
            wb = wt_ref[j % 2].astype(jnp.bfloat16)
            comm_ref[:, sub_cols(my, j)] = jnp.dot(
                xb, wb, preferred_element_type=jnp.float32
            ).astype(jnp.bfloat16)
            r1j = remote_copy(
                comm_ref.at[:, sub_cols(my, j)],
                comm_ref.at[:, sub_cols(my, j)],
                send_sems, recv_sems, j, right,
            )
            l1j = remote_copy(
                comm_ref.at[:, sub_cols(my, j)],
                comm_ref.at[:, sub_cols(my, j)],
                send_sems, recv_sems, NT + j, left,
            )
            r1j.start()
            l1j.start()
            hop1.append((r1j, l1j))

        def chunk_stats(col0, ms):
            def tbody(t, carry):
                m, s = carry
                l = comm_ref[:, pl.ds(col0 + t * TN, TN)].astype(jnp.float32)
                tm = jnp.max(l, axis=1, keepdims=True)
                te = jnp.exp(l - tm)
                ts = jnp.sum(te, axis=1, keepdims=True)
                nm = jnp.maximum(m, tm)
                return nm, s * jnp.exp(m - nm) + ts * jnp.exp(tm - nm)
            return lax.fori_loop(0, NL // TN, tbody, ms)

        ms = (
            jnp.full((M, 1), -jnp.inf, jnp.float32),
            jnp.zeros((M, 1), jnp.float32),
        )
        ms = chunk_stats(my * NL, ms)

        for r1j, _ in hop1:
            r1j.wait_recv()
        r2 = remote_copy(
            comm_ref.at[pl.ds(0, MH), cols(left)],
            comm_ref.at[pl.ds(0, MH), cols(left)],
            s2_send, s2_recv, 0, right,
        )
        r2.start()
        for _, l1j in hop1:
            l1j.wait_recv()
        l2 = remote_copy(
            comm_ref.at[pl.ds(MH, MH), cols(right)],
            comm_ref.at[pl.ds(MH, MH), cols(right)],
            s2_send, s2_recv, 1, left,
        )
        l2.start()
        for r1j, l1j in hop1:
            r1j.wait_send()
            l1j.wait_send()

        ms = chunk_stats(left * NL, ms)
        ms = chunk_stats(right * NL, ms)

        r2.wait_recv()
        l2.wait_recv()
        r2.wait_send()
        l2.wait_send()
        opp = lax.rem(my + 2, N_DEV)
        ms = chunk_stats(opp * NL, ms)

        m, s = ms
        r = 1.0 / s

        def out_dma(t):
            return pltpu.make_async_copy(
                comm_ref.at[:, pl.ds(t * TN, TN)],
                out_ref.at[:, pl.ds(t * TN, TN)],
                copy_sem,
            )

        def norm_body(t, carry):
            l = comm_ref[:, pl.ds(t * TN, TN)].astype(jnp.float32)
            comm_ref[:, pl.ds(t * TN, TN)] = (
                jnp.exp(l - m) * r
            ).astype(jnp.bfloat16)
            out_dma(t).start()
            return carry

        lax.fori_loop(0, NG // TN, norm_body, jnp.zeros((1, 1), jnp.float32))

        def drain_body(t, carry):
            out_dma(t).wait()
            return carry

        lax.fori_loop(0, NG // TN, drain_body, jnp.zeros((1, 1), jnp.float32))

    return pl.pallas_call(
        body,
        out_shape=jax.ShapeDtypeStruct((M, NG), jnp.bfloat16),
        in_specs=[
            pl.BlockSpec(memory_space=pltpu.VMEM),
            pl.BlockSpec(memory_space=pl.ANY),
        ],
        out_specs=pl.BlockSpec(memory_space=pl.ANY),
        scratch_shapes=[
            pltpu.VMEM((M, NG), jnp.bfloat16),
            pltpu.VMEM((2, K, TW), jnp.float32),
            pltpu.SemaphoreType.DMA((2,)),
            pltpu.SemaphoreType.DMA((2 * NT,)),
            pltpu.SemaphoreType.DMA((2 * NT,)),
            pltpu.SemaphoreType.DMA((2,)),
            pltpu.SemaphoreType.DMA((2,)),
            pltpu.SemaphoreType.DMA,
        ],
        compiler_params=pltpu.CompilerParams(
            collective_id=0, vmem_limit_bytes=60 * 1024 * 1024
        ),
    )(x, W)


def kernel(x, W):
    return _fused(x, W)
